# Initial kernel scaffold; baseline (speedup 1.0000x reference)
import jax
import jax.numpy as jnp
from jax.experimental import pallas as pl


def kernel(x, edge_index, edge_attr, params):
    raise NotImplementedError("write your pallas kernel here")


# speedup vs baseline: 7.3696x; 7.3696x over previous
import functools

import jax
import jax.numpy as jnp
from jax import lax
from jax.experimental import pallas as pl
from jax.experimental.pallas import tpu as pltpu
from jax.experimental.pallas import tpu_sc as plsc

U = 32
DEPTH = 12
_CH = 125
_G = 8
_NC = 2
_NS = 16
_NW = _NC * _NS


def _silu(v):
    return v * jax.nn.sigmoid(v)


def _pick_block(rows, want):
    if rows % want == 0:
        return want
    return rows


def _pack(a):
    e = a.shape[0]
    return a.reshape(e // 4, 4 * a.shape[1])


def _sc_gather(table, idx2d):
    n, u = table.shape
    nrow, ch = idx2d.shape
    e = nrow * ch
    nch_w = nrow // _NW
    g = min(_G, nch_w)
    no = nch_w // g
    mesh = plsc.VectorSubcoreMesh(core_axis_name="c", subcore_axis_name="s")

    @functools.partial(
        pl.kernel,
        out_type=jax.ShapeDtypeStruct((e, u), jnp.float32),
        mesh=mesh,
        scratch_types=[
            pltpu.VMEM((g, ch), jnp.int32),
            pltpu.VMEM((g * ch, u), jnp.float32),
            pltpu.SemaphoreType.DMA,
        ],
        compiler_params=pltpu.CompilerParams(use_tc_tiling_on_sc=False),
    )
    def k(table_hbm, idx_hbm, out_hbm, idxbuf, rows, sem):
        wid = lax.axis_index("c") * _NS + lax.axis_index("s")
        row0 = wid * nch_w

        def outer(o, carry):
            r0 = row0 + o * g
            pltpu.sync_copy(idx_hbm.at[pl.ds(r0, g)], idxbuf)
            descs = [
                pltpu.async_copy(
                    table_hbm.at[idxbuf.at[j]], rows.at[pl.ds(j * ch, ch)], sem
                )
                for j in range(g)
            ]
            for d in descs:
                d.wait()
            pltpu.sync_copy(rows, out_hbm.at[pl.ds(r0 * ch, g * ch)])
            return carry

        lax.fori_loop(0, no, outer, 0)

    return k(table, idx2d)


def _sc_scatter_half(msg, idx2d, n, h0):
    nrow, ch = idx2d.shape
    hw = U // 2
    nch_w = nrow // _NW
    g = min(_G, nch_w)
    no = nch_w // g
    nslice = n // _NS
    zchunk = next(zc for zc in (1250, 625, nslice) if nslice % zc == 0)
    nz = nslice // zchunk
    zeros = jnp.zeros((zchunk, hw), jnp.float32)
    mesh = plsc.VectorSubcoreMesh(core_axis_name="c", subcore_axis_name="s")

    @functools.partial(
        pl.kernel,
        out_type=jax.ShapeDtypeStruct((_NC, n, hw), jnp.float32),
        mesh=mesh,
        scratch_types=[
            pltpu.VMEM((g, ch), jnp.int32),
            pltpu.VMEM((g * ch, hw), jnp.float32),
            pltpu.VMEM_SHARED((n, hw), jnp.float32),
            pltpu.SemaphoreType.DMA,
        ],
        compiler_params=pltpu.CompilerParams(use_tc_tiling_on_sc=False),
    )
    def k(msg_hbm, idx_hbm, z_hbm, out_hbm, idxbuf, rows, acc, sem):
        c = lax.axis_index("c")
        s = lax.axis_index("s")
        wid = c * _NS + s

        def zloop(t, carry):
            pltpu.sync_copy(
                z_hbm, acc.at[pl.ds(s * nslice + t * zchunk, zchunk)]
            )
            return carry

        lax.fori_loop(0, nz, zloop, 0)
        plsc.subcore_barrier()
        row0 = wid * nch_w

        def outer(o, carry):
            r0 = row0 + o * g
            pltpu.sync_copy(idx_hbm.at[pl.ds(r0, g)], idxbuf)
            pltpu.sync_copy(
                msg_hbm.at[pl.ds(r0 * ch, g * ch), pl.ds(h0, hw)], rows
            )
            for j in range(g):
                pltpu.sync_copy(
                    rows.at[pl.ds(j * ch, ch)], acc.at[idxbuf.at[j]], add=True
                )
            return carry

        lax.fori_loop(0, no, outer, 0)
        plsc.subcore_barrier()
        pltpu.sync_copy(
            acc.at[pl.ds(s * nslice, nslice)],
            out_hbm.at[c, pl.ds(s * nslice, nslice)],
        )

    return k(msg, idx2d, zeros)


def _sc_scatter(msg, idx2d, n):
    lo = _sc_scatter_half(msg, idx2d, n, 0)
    hi = _sc_scatter_half(msg, idx2d, n, U // 2)
    return jnp.concatenate([lo, hi], axis=2)


def _tc_init_nodes(x, w0t, b0):
    n = x.shape[0]

    def body(x_ref, w_ref, b_ref, o_ref):
        o_ref[...] = _silu(
            jnp.dot(x_ref[...], w_ref[...], preferred_element_type=jnp.float32)
            + b_ref[...]
        )

    return pl.pallas_call(
        body,
        out_shape=jax.ShapeDtypeStruct((n, U), jnp.float32),
    )(x, w0t, b0)


def _tc_node_tables(h, cat_w, cat_b, se_w1t, se_w2t):
    n = h.shape[0]
    c = _pick_block(n, 10000)
    nsteps = n // c

    def body(h_ref, cw_ref, cb_ref, w1_ref, w2_ref, x1_ref, x2_ref, x3_ref, x4_ref):
        y = (
            jnp.dot(h_ref[...], cw_ref[...], preferred_element_type=jnp.float32)
            + cb_ref[...]
        )
        for j, out in enumerate([x1_ref, x2_ref, x3_ref, x4_ref]):
            yj = y[:, U * j:U * (j + 1)]
            q = jnp.maximum(
                jnp.dot(yj, w1_ref[...], preferred_element_type=jnp.float32), 0.0
            )
            r = jnp.dot(q, w2_ref[...], preferred_element_type=jnp.float32)
            out[...] = yj * jax.nn.sigmoid(r)

    row = pl.BlockSpec((c, U), lambda i: (i, 0))
    full = lambda shape: pl.BlockSpec(shape, lambda i: tuple(0 for _ in shape))
    return pl.pallas_call(
        body,
        grid=(nsteps,),
        in_specs=[
            pl.BlockSpec((c, U), lambda i: (i, 0)),
            full((U, 4 * U)),
            full((1, 4 * U)),
            full((U, 2)),
            full((2, U)),
        ],
        out_specs=[row, row, row, row],
        out_shape=[jax.ShapeDtypeStruct((n, U), jnp.float32)] * 4,
    )(h, cat_w, cat_b, se_w1t, se_w2t)


def _edge_specs(c, extra_in):
    row = pl.BlockSpec((c, 128), lambda i: (i, 0))
    return row


def _tc_edge0(ea4, g2p, g4p, gsp, s0, b0t, wblk, bt):
    r = ea4.shape[0]
    c = _pick_block(r, 2000)
    nsteps = r // c

    def body(a_ref, g2_ref, g4_ref, gs_ref, s_ref, b0_ref, wb_ref, bt_ref,
             w_ref, z_ref, m_ref, st_ref, acc_ref):
        i = pl.program_id(0)
        w = _silu(
            jnp.dot(a_ref[...], s_ref[...], preferred_element_type=jnp.float32)
            + b0_ref[...]
        )
        z = (
            jnp.dot(w, wb_ref[...], preferred_element_type=jnp.float32)
            + bt_ref[...] + gs_ref[...] + g4_ref[...]
        )
        w_ref[...] = w
        z_ref[...] = z
        m_ref[...] = jax.nn.sigmoid(w) * g2_ref[...]

        @pl.when(i == 0)
        def _():
            acc_ref[...] = jnp.zeros_like(acc_ref)

        acc_ref[0:1, :] += jnp.sum(z, axis=0, keepdims=True)
        acc_ref[1:2, :] += jnp.sum(z * z, axis=0, keepdims=True)

        @pl.when(i == nsteps - 1)
        def _():
            st_ref[...] = acc_ref[...]

    row = pl.BlockSpec((c, 128), lambda i: (i, 0))
    full = lambda shape: pl.BlockSpec(shape, lambda i: tuple(0 for _ in shape))
    return pl.pallas_call(
        body,
        grid=(nsteps,),
        in_specs=[
            pl.BlockSpec((c, 4), lambda i: (i, 0)),
            row, row, row,
            full((4, 128)), full((1, 128)), full((128, 128)), full((1, 128)),
        ],
        out_specs=[row, row, row, full((2, 128))],
        out_shape=[
            jax.ShapeDtypeStruct((r, 128), jnp.float32),
            jax.ShapeDtypeStruct((r, 128), jnp.float32),
            jax.ShapeDtypeStruct((r, 128), jnp.float32),
            jax.ShapeDtypeStruct((2, 128), jnp.float32),
        ],
        scratch_shapes=[pltpu.VMEM((2, 128), jnp.float32)],
    )(ea4, g2p, g4p, gsp, s0, b0t, wblk, bt)


def _tc_edge(wprev, zprev, ss, g2p, g4p, gsp, wblk, bt):
    r = wprev.shape[0]
    c = _pick_block(r, 2000)
    nsteps = r // c

    def body(wp_ref, zp_ref, ss_ref, g2_ref, g4_ref, gs_ref, wb_ref, bt_ref,
             w_ref, z_ref, m_ref, st_ref, acc_ref):
        i = pl.program_id(0)
        w = wp_ref[...] + _silu(zp_ref[...] * ss_ref[0:1, :] + ss_ref[1:2, :])
        z = (
            jnp.dot(w, wb_ref[...], preferred_element_type=jnp.float32)
            + bt_ref[...] + gs_ref[...] + g4_ref[...]
        )
        w_ref[...] = w
        z_ref[...] = z
        m_ref[...] = jax.nn.sigmoid(w) * g2_ref[...]

        @pl.when(i == 0)
        def _():
            acc_ref[...] = jnp.zeros_like(acc_ref)

        acc_ref[0:1, :] += jnp.sum(z, axis=0, keepdims=True)
        acc_ref[1:2, :] += jnp.sum(z * z, axis=0, keepdims=True)

        @pl.when(i == nsteps - 1)
        def _():
            st_ref[...] = acc_ref[...]

    row = pl.BlockSpec((c, 128), lambda i: (i, 0))
    full = lambda shape: pl.BlockSpec(shape, lambda i: tuple(0 for _ in shape))
    return pl.pallas_call(
        body,
        grid=(nsteps,),
        in_specs=[
            row, row, full((2, 128)),
            row, row, row,
            full((128, 128)), full((1, 128)),
        ],
        out_specs=[row, row, row, full((2, 128))],
        out_shape=[
            jax.ShapeDtypeStruct((r, 128), jnp.float32),
            jax.ShapeDtypeStruct((r, 128), jnp.float32),
            jax.ShapeDtypeStruct((r, 128), jnp.float32),
            jax.ShapeDtypeStruct((2, 128), jnp.float32),
        ],
        scratch_shapes=[pltpu.VMEM((2, 128), jnp.float32)],
    )(wprev, zprev, ss, g2p, g4p, gsp, wblk, bt)


def _tc_update(h, x1, acc0, acc1, cacc0, cacc1, st, e_count,
               vg, vb, eg, eb):
    n = h.shape[0]
    c = _pick_block(n, 5000)
    nsteps = n // c
    row = pl.BlockSpec((c, U), lambda i: (i, 0))
    rowh = pl.BlockSpec((c, U // 2), lambda i: (i, 0))
    full = lambda shape: pl.BlockSpec(shape, lambda i: tuple(0 for _ in shape))

    def _y(x1_ref, a0_ref, a1_ref, c0_ref, c1_ref):
        cnt = jnp.maximum(c0_ref[:, 0:1] + c1_ref[:, 0:1], 1.0)
        return x1_ref[...] + (a0_ref[...] + a1_ref[...]) / cnt

    def body_a(x1_ref, a0_ref, a1_ref, c0_ref, c1_ref, nst_ref, acc_ref):
        i = pl.program_id(0)
        y = _y(x1_ref, a0_ref, a1_ref, c0_ref, c1_ref)

        @pl.when(i == 0)
        def _():
            acc_ref[...] = jnp.zeros_like(acc_ref)

        acc_ref[0:1, :] += jnp.sum(y, axis=0, keepdims=True)
        acc_ref[1:2, :] += jnp.sum(y * y, axis=0, keepdims=True)

        @pl.when(i == nsteps - 1)
        def _():
            nst_ref[...] = acc_ref[...]

    nst = pl.pallas_call(
        body_a,
        grid=(nsteps,),
        in_specs=[row, row, row, rowh, rowh],
        out_specs=full((2, U)),
        out_shape=jax.ShapeDtypeStruct((2, U), jnp.float32),
        scratch_shapes=[pltpu.VMEM((2, U), jnp.float32)],
    )(x1, acc0, acc1, cacc0, cacc1)

    def body_b(h_ref, x1_ref, a0_ref, a1_ref, c0_ref, c1_ref, nst_ref, st_ref,
               vg_ref, vb_ref, eg_ref, eb_ref, hn_ref, ss_ref):
        i = pl.program_id(0)
        y = _y(x1_ref, a0_ref, a1_ref, c0_ref, c1_ref)
        nm = nst_ref[0:1, :] / float(n)
        nv = nst_ref[1:2, :] / float(n) - nm * nm
        bn = vg_ref[...] * (y - nm) * jax.lax.rsqrt(nv + 1e-5) + vb_ref[...]
        hn_ref[...] = h_ref[...] + _silu(bn)

        @pl.when(i == 0)
        def _():
            st = st_ref[...]
            s32 = sum(st[0:1, U * j:U * (j + 1)] for j in range(4))
            q32 = sum(st[1:2, U * j:U * (j + 1)] for j in range(4))
            em = s32 / e_count
            ev = q32 / e_count - em * em
            scale = eg_ref[...] * jax.lax.rsqrt(ev + 1e-5)
            shift = eb_ref[...] - em * scale
            ss_ref[...] = jnp.concatenate(
                [jnp.tile(scale, (1, 4)), jnp.tile(shift, (1, 4))], axis=0
            )

    return pl.pallas_call(
        body_b,
        grid=(nsteps,),
        in_specs=[row, row, row, row, rowh, rowh,
                  full((2, U)), full((2, 128)),
                  full((1, U)), full((1, U)), full((1, U)), full((1, U))],
        out_specs=[row, full((2, 128))],
        out_shape=[
            jax.ShapeDtypeStruct((n, U), jnp.float32),
            jax.ShapeDtypeStruct((2, 128), jnp.float32),
        ],
    )(h, x1, acc0, acc1, cacc0, cacc1, nst, st, vg, vb, eg, eb)


def _tc_head(wprev, zprev, ss, p0blk, b0t, p1blk, b1t, p2blk, b2t):
    r = wprev.shape[0]
    c = _pick_block(r, 2000)
    nsteps = r // c

    def body(wp_ref, zp_ref, ss_ref, p0_ref, b0_ref, p1_ref, b1_ref,
             p2_ref, b2_ref, o_ref):
        w = wp_ref[...] + _silu(zp_ref[...] * ss_ref[0:1, :] + ss_ref[1:2, :])
        t = _silu(
            jnp.dot(w, p0_ref[...], preferred_element_type=jnp.float32)
            + b0_ref[...]
        )
        t = _silu(
            jnp.dot(t, p1_ref[...], preferred_element_type=jnp.float32)
            + b1_ref[...]
        )
        o_ref[...] = jax.nn.sigmoid(
            jnp.dot(t, p2_ref[...], preferred_element_type=jnp.float32)
            + b2_ref[...]
        )

    row = pl.BlockSpec((c, 128), lambda i: (i, 0))
    full = lambda shape: pl.BlockSpec(shape, lambda i: tuple(0 for _ in shape))
    return pl.pallas_call(
        body,
        grid=(nsteps,),
        in_specs=[
            row, row, full((2, 128)),
            full((128, 128)), full((1, 128)),
            full((128, 128)), full((1, 128)),
            full((128, 4)), full((1, 4)),
        ],
        out_specs=pl.BlockSpec((c, 4), lambda i: (i, 0)),
        out_shape=jax.ShapeDtypeStruct((r, 4), jnp.float32),
    )(wprev, zprev, ss, p0blk, b0t, p1blk, b1t, p2blk, b2t)


def kernel(x, edge_index, edge_attr, params):
    n = x.shape[0]
    e = edge_attr.shape[0]
    src = edge_index[0].astype(jnp.int32)
    dst = edge_index[1].astype(jnp.int32)
    src2d = src.reshape(e // _CH, _CH)
    dst2d = dst.reshape(e // _CH, _CH)
    ea4 = edge_attr.reshape(e // 4, 4)

    i4 = jnp.eye(4, dtype=jnp.float32)
    wblk = jax.vmap(lambda m: jnp.kron(i4, m))(
        params['e0_w'].transpose(0, 2, 1)
    )
    bt = jnp.tile(params['e0_b'], (1, 4)).reshape(DEPTH, 1, 128)
    s0 = jnp.kron(i4, params['e_lin0_w'][:, 0][None, :])
    b0t = jnp.tile(params['e_lin0_b'], 4)[None, :]

    cat_w = jnp.concatenate(
        [params[k].transpose(0, 2, 1) for k in ('v1_w', 'v2_w', 'v3_w', 'v4_w')],
        axis=2,
    )
    cat_b = jnp.concatenate(
        [params[k] for k in ('v1_b', 'v2_b', 'v3_b', 'v4_b')], axis=1
    ).reshape(DEPTH, 1, 128)
    se_w1t = params['se_w1'].T
    se_w2t = params['se_w2'].T

    p0blk = jnp.kron(i4, params['p0_w'].T)
    p1blk = jnp.kron(i4, params['p1_w'].T)
    p2blk = jnp.kron(i4, params['p2_w'].T)
    hb0 = jnp.tile(params['p0_b'], 4)[None, :]
    hb1 = jnp.tile(params['p1_b'], 4)[None, :]
    hb2 = jnp.tile(params['p2_b'], 4)[None, :]

    h = _tc_init_nodes(x, params['v_lin0_w'].T, params['v_lin0_b'][None, :])

    ones = jnp.ones((e, U), jnp.float32)
    cacc = _sc_scatter_half(ones, src2d, n, 0)
    cacc0, cacc1 = cacc[0], cacc[1]

    wprev = zprev = ss = None
    for i in range(DEPTH):
        x1, x2, x3, x4 = _tc_node_tables(
            h, cat_w[i], cat_b[i], se_w1t, se_w2t
        )
        g2 = _sc_gather(x2, dst2d)
        g4 = _sc_gather(x4, dst2d)
        gs = _sc_gather(x3, src2d)
        g2p, g4p, gsp = _pack(g2), _pack(g4), _pack(gs)
        if i == 0:
            wcur, zcur, msg, st = _tc_edge0(
                ea4, g2p, g4p, gsp, s0, b0t, wblk[0], bt[0]
            )
        else:
            wcur, zcur, msg, st = _tc_edge(
                wprev, zprev, ss, g2p, g4p, gsp, wblk[i], bt[i]
            )
        acc = _sc_scatter(msg.reshape(e, U), src2d, n)
        h, ss = _tc_update(
            h, x1, acc[0], acc[1], cacc0, cacc1, st, float(e),
            params['vbn_g'][i][None, :], params['vbn_b'][i][None, :],
            params['ebn_g'][i][None, :], params['ebn_b'][i][None, :],
        )
        wprev, zprev = wcur, zcur

    out4 = _tc_head(wprev, zprev, ss, p0blk, hb0, p1blk, hb1, p2blk, hb2)
    return out4.reshape(e)
